# baseline (speedup 1.0000x reference)
import functools

import jax
import jax.numpy as jnp
from jax.experimental import pallas as pl
from jax.experimental.pallas import tpu as pltpu
from jax.experimental.pallas import tpu_sc as plsc

N = 50000
D = 128
FE = 9
NF = 128
G = 32
EPS = 1e-5

ROW_TILE = 5000
HALF_TILE = 5000
MM_TILE = 2000
GATHER_WINDOW = 128
GATHER_BATCH = 4
T1 = 4
FLAT = FE * N
FLAT_PAD = -(-FLAT // GATHER_WINDOW) * GATHER_WINDOW


def _stats_body(lv_ref, g_ref, b_ref, scale_ref, shift_ref, acc_ref):
    i = pl.program_id(0)

    @pl.when(i == 0)
    def _():
        acc_ref[...] = jnp.zeros_like(acc_ref)

    x = lv_ref[...]
    acc_ref[0:1, :] += jnp.sum(x, axis=0, keepdims=True)
    acc_ref[1:2, :] += jnp.sum(x * x, axis=0, keepdims=True)

    @pl.when(i == pl.num_programs(0) - 1)
    def _():
        r = jax.lax.broadcasted_iota(jnp.int32, (D, D), 0) // (D // G)
        c = jax.lax.broadcasted_iota(jnp.int32, (D, D), 1) // (D // G)
        m = (r == c).astype(jnp.float32)
        gs = jnp.dot(acc_ref[0:1, :], m, preferred_element_type=jnp.float32)
        gsq = jnp.dot(acc_ref[1:2, :], m, preferred_element_type=jnp.float32)
        cnt = float((D // G) * N)
        mean = gs / cnt
        var = gsq / cnt - mean * mean
        rstd = jax.lax.rsqrt(var + EPS)
        sc = g_ref[...] * rstd
        scale_ref[...] = sc
        shift_ref[...] = b_ref[...] - mean * sc


def _pack_pairs(y):
    lo = jax.lax.bitcast_convert_type(
        y[:, : D // 2].astype(jnp.bfloat16), jnp.uint16
    ).astype(jnp.uint32)
    hi = jax.lax.bitcast_convert_type(
        y[:, D // 2 :].astype(jnp.bfloat16), jnp.uint16
    ).astype(jnp.uint32)
    return jax.lax.bitcast_convert_type(lo | (hi << 16), jnp.float32)


def _norm_body(lv_a_ref, lv_b_ref, scale_ref, shift_ref, o_ref):
    s, t = scale_ref[...], shift_ref[...]
    ya = jnp.maximum(lv_a_ref[...] * s + t, 0.0)
    yb = jnp.maximum(lv_b_ref[...] * s + t, 0.0)
    o_ref[...] = jnp.concatenate([_pack_pairs(ya), _pack_pairs(yb)], axis=1)


def _make_mm_body(ntaps, first):
    def body(*refs):
        r_refs = refs[:ntaps]
        w_ref, init_ref, o_ref = refs[ntaps:]
        if first:
            acc = jnp.broadcast_to(init_ref[...], (o_ref.shape[0], 2 * NF))
        else:
            acc = init_ref[...]
        for f in range(ntaps):
            u = jax.lax.bitcast_convert_type(r_refs[f][...], jnp.uint32)
            lo = jax.lax.bitcast_convert_type(
                (u & 0xFFFF).astype(jnp.uint16), jnp.bfloat16
            )
            hi = jax.lax.bitcast_convert_type(
                (u >> 16).astype(jnp.uint16), jnp.bfloat16
            )
            reo = jnp.concatenate([lo, hi], axis=1)
            acc = acc + jnp.dot(reo, w_ref[f], preferred_element_type=jnp.float32)
        o_ref[...] = acc.reshape(o_ref.shape)
    return body


def _sc_gather(lv_r, idx_flat):
    total = idx_flat.shape[0]
    dcols = lv_r.shape[1]
    idx2 = idx_flat.reshape(total // GATHER_WINDOW, GATHER_WINDOW)
    mesh = plsc.VectorSubcoreMesh(core_axis_name="c", subcore_axis_name="s")

    @functools.partial(
        pl.kernel,
        out_type=jax.ShapeDtypeStruct((total, dcols), lv_r.dtype),
        mesh=mesh,
        compiler_params=pltpu.CompilerParams(use_tc_tiling_on_sc=False),
    )
    def gk(x_hbm, i_hbm, o_hbm):
        def body(i_vmem, o_vmem):
            for bk in range(GATHER_BATCH):
                pltpu.sync_copy(
                    x_hbm.at[i_vmem.at[bk]],
                    o_vmem.at[pl.ds(bk * GATHER_WINDOW, GATHER_WINDOW)],
                )

        pltpu.emit_pipeline(
            body,
            grid=(total // (GATHER_WINDOW * GATHER_BATCH),),
            in_specs=[pl.BlockSpec((GATHER_BATCH, GATHER_WINDOW), lambda i: (i, 0))],
            out_specs=[
                pl.BlockSpec((GATHER_WINDOW * GATHER_BATCH, dcols), lambda i: (i, 0))
            ],
            core_axis_name=("c", "s"),
            dimension_semantics=(pltpu.PARALLEL,),
        )(i_hbm, o_hbm)

    return gk(lv_r, idx2)


def kernel(lv, neighbor_idx, gamma, beta, W, b):
    nt = N // ROW_TILE

    scale, shift = pl.pallas_call(
        _stats_body,
        grid=(nt,),
        in_specs=[
            pl.BlockSpec((ROW_TILE, D), lambda i: (i, 0)),
            pl.BlockSpec((1, D), lambda i: (0, 0)),
            pl.BlockSpec((1, D), lambda i: (0, 0)),
        ],
        out_specs=[
            pl.BlockSpec((1, D), lambda i: (0, 0)),
            pl.BlockSpec((1, D), lambda i: (0, 0)),
        ],
        out_shape=[
            jax.ShapeDtypeStruct((1, D), jnp.float32),
            jax.ShapeDtypeStruct((1, D), jnp.float32),
        ],
        scratch_shapes=[pltpu.VMEM((2, D), jnp.float32)],
    )(lv, gamma.reshape(1, D), beta.reshape(1, D))

    hb = N // 2 // HALF_TILE
    lv_r = pl.pallas_call(
        _norm_body,
        grid=(hb,),
        in_specs=[
            pl.BlockSpec((HALF_TILE, D), lambda i: (i, 0)),
            pl.BlockSpec((HALF_TILE, D), lambda i: (i + hb, 0)),
            pl.BlockSpec((1, D), lambda i: (0, 0)),
            pl.BlockSpec((1, D), lambda i: (0, 0)),
        ],
        out_specs=pl.BlockSpec((HALF_TILE, D), lambda i: (i, 0)),
        out_shape=jax.ShapeDtypeStruct((N // 2, D), jnp.float32),
    )(lv, lv, scale, shift)
    lv_r = lv_r.reshape(N, D // 2)

    idx = neighbor_idx.astype(jnp.int32)
    vrow = jnp.where(idx < N // 2, 2 * idx, 2 * idx - (N - 1))
    idx_flat = vrow.T.reshape(FLAT)

    step = GATHER_WINDOW * GATHER_BATCH

    def padded_gather(sl):
        pad_len = -len(sl) % step
        pad = jnp.arange(pad_len, dtype=jnp.int32)
        return _sc_gather(lv_r, jnp.concatenate([sl, pad]))

    rows_a = padded_gather(idx_flat[: T1 * N])
    rows_b = padded_gather(idx_flat[T1 * N :])

    rows_a2 = rows_a.reshape(rows_a.shape[0] // 2, D)
    rows_b2 = rows_b.reshape(rows_b.shape[0] // 2, D)
    Wf = W.reshape(FE, D, NF).astype(jnp.bfloat16)
    Wlo, Whi = Wf[:, : D // 2, :], Wf[:, D // 2 :, :]
    Z = jnp.zeros_like(Wlo)
    Wr = jnp.concatenate(
        [
            jnp.concatenate([Wlo, Z], axis=2),
            jnp.concatenate([Z, Wlo], axis=2),
            jnp.concatenate([Whi, Z], axis=2),
            jnp.concatenate([Z, Whi], axis=2),
        ],
        axis=1,
    )
    mt = N // MM_TILE

    def row_specs(ntaps):
        return [
            pl.BlockSpec(
                (MM_TILE // 2, D),
                functools.partial(lambda f, i: (f * mt + i, 0), f),
            )
            for f in range(ntaps)
        ]

    partial_out = pl.pallas_call(
        _make_mm_body(T1, True),
        grid=(mt,),
        in_specs=row_specs(T1) + [
            pl.BlockSpec((T1, 2 * D, 2 * NF), lambda i: (0, 0, 0)),
            pl.BlockSpec((1, 2 * NF), lambda i: (0, 0)),
        ],
        out_specs=pl.BlockSpec((MM_TILE // 2, 2 * NF), lambda i: (i, 0)),
        out_shape=jax.ShapeDtypeStruct((N // 2, 2 * NF), jnp.float32),
    )(*([rows_a2] * T1), Wr[:T1], jnp.concatenate([b, b]).reshape(1, 2 * NF))

    t2 = FE - T1
    out = pl.pallas_call(
        _make_mm_body(t2, False),
        grid=(mt,),
        in_specs=row_specs(t2) + [
            pl.BlockSpec((t2, 2 * D, 2 * NF), lambda i: (0, 0, 0)),
            pl.BlockSpec((MM_TILE // 2, 2 * NF), lambda i: (i, 0)),
        ],
        out_specs=pl.BlockSpec((MM_TILE, NF), lambda i: (i, 0)),
        out_shape=jax.ShapeDtypeStruct((N, NF), jnp.float32),
    )(*([rows_b2] * t2), Wr[T1:], partial_out)

    return out


# speedup vs baseline: 1.0450x; 1.0450x over previous
import functools

import jax
import jax.numpy as jnp
from jax.experimental import pallas as pl
from jax.experimental.pallas import tpu as pltpu
from jax.experimental.pallas import tpu_sc as plsc

N = 50000
D = 128
FE = 9
NF = 128
G = 32
EPS = 1e-5

ROW_TILE = 5000
HALF_TILE = 5000
MM_TILE = 2000
GATHER_WINDOW = 128
GATHER_BATCH = 4
FLAT = FE * N
FLAT_PAD = -(-FLAT // GATHER_WINDOW) * GATHER_WINDOW


def _stats_body(lv_ref, g_ref, b_ref, scale_ref, shift_ref, acc_ref):
    i = pl.program_id(0)

    @pl.when(i == 0)
    def _():
        acc_ref[...] = jnp.zeros_like(acc_ref)

    x = lv_ref[...]
    acc_ref[0:1, :] += jnp.sum(x, axis=0, keepdims=True)
    acc_ref[1:2, :] += jnp.sum(x * x, axis=0, keepdims=True)

    @pl.when(i == pl.num_programs(0) - 1)
    def _():
        r = jax.lax.broadcasted_iota(jnp.int32, (D, D), 0) // (D // G)
        c = jax.lax.broadcasted_iota(jnp.int32, (D, D), 1) // (D // G)
        m = (r == c).astype(jnp.float32)
        gs = jnp.dot(acc_ref[0:1, :], m, preferred_element_type=jnp.float32)
        gsq = jnp.dot(acc_ref[1:2, :], m, preferred_element_type=jnp.float32)
        cnt = float((D // G) * N)
        mean = gs / cnt
        var = gsq / cnt - mean * mean
        rstd = jax.lax.rsqrt(var + EPS)
        sc = g_ref[...] * rstd
        scale_ref[...] = sc
        shift_ref[...] = b_ref[...] - mean * sc


def _pack_pairs(y):
    lo = jax.lax.bitcast_convert_type(
        y[:, : D // 2].astype(jnp.bfloat16), jnp.uint16
    ).astype(jnp.uint32)
    hi = jax.lax.bitcast_convert_type(
        y[:, D // 2 :].astype(jnp.bfloat16), jnp.uint16
    ).astype(jnp.uint32)
    return jax.lax.bitcast_convert_type(lo | (hi << 16), jnp.float32)


def _norm_body(lv_a_ref, lv_b_ref, scale_ref, shift_ref, o_ref):
    s, t = scale_ref[...], shift_ref[...]
    ya = jnp.maximum(lv_a_ref[...] * s + t, 0.0)
    yb = jnp.maximum(lv_b_ref[...] * s + t, 0.0)
    o_ref[...] = jnp.concatenate([_pack_pairs(ya), _pack_pairs(yb)], axis=1)


def _mm_body(*refs):
    r_refs, (w_ref, b_ref, o_ref) = refs[:FE], refs[FE:]
    acc = jnp.broadcast_to(b_ref[...], (o_ref.shape[0] // 2, 2 * NF))
    for f in range(FE):
        u = jax.lax.bitcast_convert_type(r_refs[f][...], jnp.uint32)
        lo = jax.lax.bitcast_convert_type(
            (u & 0xFFFF).astype(jnp.uint16), jnp.bfloat16
        )
        hi = jax.lax.bitcast_convert_type(
            (u >> 16).astype(jnp.uint16), jnp.bfloat16
        )
        reo = jnp.concatenate([lo, hi], axis=1)
        acc = acc + jnp.dot(reo, w_ref[f], preferred_element_type=jnp.float32)
    o_ref[...] = acc.reshape(o_ref.shape)


def _sc_gather(lv_r, idx_flat):
    total = idx_flat.shape[0]
    dcols = lv_r.shape[1]
    idx2 = idx_flat.reshape(total // GATHER_WINDOW, GATHER_WINDOW)
    mesh = plsc.VectorSubcoreMesh(core_axis_name="c", subcore_axis_name="s")

    @functools.partial(
        pl.kernel,
        out_type=jax.ShapeDtypeStruct((total, dcols), lv_r.dtype),
        mesh=mesh,
        compiler_params=pltpu.CompilerParams(use_tc_tiling_on_sc=False),
    )
    def gk(x_hbm, i_hbm, o_hbm):
        def body(i_vmem, o_vmem):
            for bk in range(GATHER_BATCH):
                pltpu.sync_copy(
                    x_hbm.at[i_vmem.at[bk]],
                    o_vmem.at[pl.ds(bk * GATHER_WINDOW, GATHER_WINDOW)],
                )

        pltpu.emit_pipeline(
            body,
            grid=(total // (GATHER_WINDOW * GATHER_BATCH),),
            in_specs=[pl.BlockSpec((GATHER_BATCH, GATHER_WINDOW), lambda i: (i, 0))],
            out_specs=[
                pl.BlockSpec((GATHER_WINDOW * GATHER_BATCH, dcols), lambda i: (i, 0))
            ],
            core_axis_name=("c", "s"),
            dimension_semantics=(pltpu.PARALLEL,),
        )(i_hbm, o_hbm)

    return gk(lv_r, idx2)


def kernel(lv, neighbor_idx, gamma, beta, W, b):
    nt = N // ROW_TILE

    scale, shift = pl.pallas_call(
        _stats_body,
        grid=(nt,),
        in_specs=[
            pl.BlockSpec((ROW_TILE, D), lambda i: (i, 0)),
            pl.BlockSpec((1, D), lambda i: (0, 0)),
            pl.BlockSpec((1, D), lambda i: (0, 0)),
        ],
        out_specs=[
            pl.BlockSpec((1, D), lambda i: (0, 0)),
            pl.BlockSpec((1, D), lambda i: (0, 0)),
        ],
        out_shape=[
            jax.ShapeDtypeStruct((1, D), jnp.float32),
            jax.ShapeDtypeStruct((1, D), jnp.float32),
        ],
        scratch_shapes=[pltpu.VMEM((2, D), jnp.float32)],
    )(lv, gamma.reshape(1, D), beta.reshape(1, D))

    hb = N // 2 // HALF_TILE
    lv_r = pl.pallas_call(
        _norm_body,
        grid=(hb,),
        in_specs=[
            pl.BlockSpec((HALF_TILE, D), lambda i: (i, 0)),
            pl.BlockSpec((HALF_TILE, D), lambda i: (i + hb, 0)),
            pl.BlockSpec((1, D), lambda i: (0, 0)),
            pl.BlockSpec((1, D), lambda i: (0, 0)),
        ],
        out_specs=pl.BlockSpec((HALF_TILE, D), lambda i: (i, 0)),
        out_shape=jax.ShapeDtypeStruct((N // 2, D), jnp.float32),
    )(lv, lv, scale, shift)
    lv_r = lv_r.reshape(N, D // 2)

    idx = neighbor_idx.astype(jnp.int32)
    vrow = jnp.where(idx < N // 2, 2 * idx, 2 * idx - (N - 1))
    idx_flat = vrow.T.reshape(FLAT)
    pad = jnp.arange(FLAT_PAD - FLAT, dtype=jnp.int32)
    rows = _sc_gather(lv_r, jnp.concatenate([idx_flat, pad]))

    rows2 = rows.reshape(FLAT_PAD // 2, D)
    Wf = W.reshape(FE, D, NF).astype(jnp.bfloat16)
    Wlo, Whi = Wf[:, : D // 2, :], Wf[:, D // 2 :, :]
    Z = jnp.zeros_like(Wlo)
    Wr = jnp.concatenate(
        [
            jnp.concatenate([Wlo, Z], axis=2),
            jnp.concatenate([Z, Wlo], axis=2),
            jnp.concatenate([Whi, Z], axis=2),
            jnp.concatenate([Z, Whi], axis=2),
        ],
        axis=1,
    )
    mt = N // MM_TILE
    row_specs = [
        pl.BlockSpec(
            (MM_TILE // 2, D),
            functools.partial(lambda f, i: (f * mt + i, 0), f),
        )
        for f in range(FE)
    ]
    out = pl.pallas_call(
        _mm_body,
        grid=(mt,),
        in_specs=row_specs + [
            pl.BlockSpec((FE, 2 * D, 2 * NF), lambda i: (0, 0, 0)),
            pl.BlockSpec((1, 2 * NF), lambda i: (0, 0)),
        ],
        out_specs=pl.BlockSpec((MM_TILE, NF), lambda i: (i, 0)),
        out_shape=jax.ShapeDtypeStruct((N, NF), jnp.float32),
    )(*([rows2] * FE), Wr, jnp.concatenate([b, b]).reshape(1, 2 * NF))

    return out
